# Initial kernel scaffold; baseline (speedup 1.0000x reference)
import jax
import jax.numpy as jnp
from jax.experimental import pallas as pl


def kernel(x, edge_index, batch_idx, conv_Ws, skip_Ws, mlp_W1, mlp_b1, mlp_W2, mlp_b2):
    raise NotImplementedError("write your pallas kernel here")


# speedup vs baseline: 1.8955x; 1.8955x over previous
import functools

import jax
import jax.numpy as jnp
from jax import lax
from jax.experimental import pallas as pl
from jax.experimental.pallas import tpu as pltpu
from jax.experimental.pallas import tpu_sc as plsc

F32 = jnp.float32
NUM_SC = 2
NUM_TILES = 16
K_EDGE = 128


def _seg_sum_sc(h_blk, src_p, dst_p):
    n_cb, n_nodes, cbw = h_blk.shape
    e_pad = src_p.shape[0]
    e_half = e_pad // NUM_SC
    e_tile = e_half // NUM_TILES
    n_batches = e_tile // K_EDGE
    assert e_tile % K_EDGE == 0
    nacc = n_nodes + 1
    nacc += (-nacc) % (NUM_TILES * 8)
    rows_per_tile = nacc // NUM_TILES
    chunks = []
    off = 0
    while off < rows_per_tile:
        ln = min(128, rows_per_tile - off)
        chunks.append((off, ln))
        off += ln

    mesh = plsc.VectorSubcoreMesh(core_axis_name="c", subcore_axis_name="s")
    out_type = jax.ShapeDtypeStruct((NUM_SC, n_cb, nacc, cbw), F32)
    scratch = [
        pltpu.VMEM((K_EDGE,), jnp.int32),
        pltpu.VMEM((K_EDGE,), jnp.int32),
        pltpu.VMEM((128, cbw), F32),
        pltpu.VMEM((128, cbw), F32),
        pltpu.VMEM_SHARED((nacc, cbw), F32),
        pltpu.SemaphoreType.DMA,
    ]

    def body(h_hbm, src_hbm, dst_hbm, out_hbm, sidx, didx, rowbuf, zbuf,
             acc, sem):
        cid = lax.axis_index("c")
        sid = lax.axis_index("s")
        tile = cid * NUM_TILES + sid
        rbase = sid * rows_per_tile
        ebase = tile * e_tile

        z16 = jnp.zeros((16,), F32)
        def zrow(r, _):
            for j in range(cbw // 16):
                zbuf[r, pl.ds(j * 16, 16)] = z16
            return 0
        lax.fori_loop(0, 128, zrow, 0)

        for cb in range(n_cb):
            for c0, ln in chunks:
                pltpu.sync_copy(zbuf.at[pl.ds(0, ln)],
                                acc.at[pl.ds(rbase + c0, ln)])
            plsc.subcore_barrier()

            def step(t, _):
                off = ebase + t * K_EDGE
                pltpu.sync_copy(src_hbm.at[pl.ds(off, K_EDGE)], sidx)
                pltpu.sync_copy(dst_hbm.at[pl.ds(off, K_EDGE)], didx)
                pltpu.async_copy(h_hbm.at[cb].at[sidx], rowbuf.at[pl.ds(0, K_EDGE)], sem).wait()
                pltpu.sync_copy(rowbuf.at[pl.ds(0, K_EDGE)], acc.at[didx],
                                add=True)
                return 0
            lax.fori_loop(0, n_batches, step, 0)
            plsc.subcore_barrier()

            for c0, ln in chunks:
                r0 = rbase + c0
                pltpu.sync_copy(acc.at[pl.ds(r0, ln)], rowbuf.at[pl.ds(0, ln)])
                pltpu.sync_copy(rowbuf.at[pl.ds(0, ln)],
                                out_hbm.at[cid, cb].at[pl.ds(r0, ln)])
            plsc.subcore_barrier()

    f = pl.kernel(body, out_type=out_type, mesh=mesh, scratch_types=scratch)
    return f(h_blk, src_p, dst_p)


def _deg_sc(dst_p, n_nodes):
    e_pad = dst_p.shape[0]
    e_tile = e_pad // (NUM_SC * NUM_TILES)
    n_batches = e_tile // K_EDGE
    nacc = n_nodes + 1
    nacc += (-nacc) % (NUM_TILES * 8)
    rows_per_tile = nacc // NUM_TILES
    chunks = []
    off = 0
    while off < rows_per_tile:
        ln = min(128, rows_per_tile - off)
        chunks.append((off, ln))
        off += ln

    mesh = plsc.VectorSubcoreMesh(core_axis_name="c", subcore_axis_name="s")
    out_type = jax.ShapeDtypeStruct((NUM_SC, nacc, 128), F32)
    scratch = [
        pltpu.VMEM((K_EDGE,), jnp.int32),
        pltpu.VMEM((K_EDGE, 128), F32),
        pltpu.VMEM((128, 128), F32),
        pltpu.VMEM_SHARED((nacc, 128), F32),
    ]

    def body(dst_hbm, deg_hbm, didx, ones16, dzbuf, dacc):
        cid = lax.axis_index("c")
        sid = lax.axis_index("s")
        tile = cid * NUM_TILES + sid
        rbase = sid * rows_per_tile
        ebase = tile * e_tile

        z16 = jnp.zeros((16,), F32)
        hot = jnp.where(lax.iota(jnp.int32, 16) == 0, 1.0, 0.0).astype(F32)
        def orow(r, _):
            ones16[r, pl.ds(0, 16)] = hot
            for j in range(1, 8):
                ones16[r, pl.ds(j * 16, 16)] = z16
            for j in range(8):
                dzbuf[r, pl.ds(j * 16, 16)] = z16
            return 0
        lax.fori_loop(0, K_EDGE, orow, 0)

        for c0, ln in chunks:
            pltpu.sync_copy(dzbuf.at[pl.ds(0, ln)],
                            dacc.at[pl.ds(rbase + c0, ln)])
        plsc.subcore_barrier()

        def step(t, _):
            off = ebase + t * K_EDGE
            pltpu.sync_copy(dst_hbm.at[pl.ds(off, K_EDGE)], didx)
            pltpu.sync_copy(ones16, dacc.at[didx], add=True)
            return 0
        lax.fori_loop(0, n_batches, step, 0)
        plsc.subcore_barrier()

        for c0, ln in chunks:
            r0 = rbase + c0
            pltpu.sync_copy(dacc.at[pl.ds(r0, ln)], dzbuf.at[pl.ds(0, ln)])
            pltpu.sync_copy(dzbuf.at[pl.ds(0, ln)],
                            deg_hbm.at[cid].at[pl.ds(r0, ln)])
        plsc.subcore_barrier()

    f = pl.kernel(body, out_type=out_type, mesh=mesh, scratch_types=scratch)
    return f(dst_p)


def _tc_layer(gpair, h_blk, deg_pair, w, ws):
    n_in, n_nodes, cbin = h_blk.shape
    cin, cout = w.shape
    cbout = min(cout, 128)
    n_out = cout // cbout
    bn = 1000
    n_blk = n_nodes // bn
    grid = (n_out, n_blk, n_in)

    def body(g_ref, h_ref, d_ref, w_ref, ws_ref, o_ref):
        k = pl.program_id(2)
        deg = d_ref[0, :, 0:1] + d_ref[1, :, 0:1]
        invd = 1.0 / jnp.maximum(deg, 1.0)
        g = (g_ref[0, 0] + g_ref[1, 0]) * invd
        part = jnp.dot(g, w_ref[...], preferred_element_type=F32)
        part += jnp.dot(h_ref[0], ws_ref[...], preferred_element_type=F32)

        @pl.when(k == 0)
        def _():
            o_ref[0] = part

        @pl.when(k != 0)
        def _():
            o_ref[0] += part

        @pl.when(k == n_in - 1)
        def _():
            o_ref[0] = jnp.maximum(o_ref[0], 0.0)

    return pl.pallas_call(
        body,
        grid=grid,
        in_specs=[
            pl.BlockSpec((2, 1, bn, cbin), lambda o, i, k: (0, k, i, 0)),
            pl.BlockSpec((1, bn, cbin), lambda o, i, k: (k, i, 0)),
            pl.BlockSpec((2, bn, 128), lambda o, i, k: (0, i, 0)),
            pl.BlockSpec((cbin, cbout), lambda o, i, k: (k, o)),
            pl.BlockSpec((cbin, cbout), lambda o, i, k: (k, o)),
        ],
        out_specs=pl.BlockSpec((1, bn, cbout), lambda o, i, k: (o, i, 0)),
        out_shape=jax.ShapeDtypeStruct((n_out, n_nodes, cbout), F32),
    )(gpair, h_blk, deg_pair, w, ws)


def _tc_pool(h_blk, bidx3, n_batch):
    n_cb, n_nodes, cbw = h_blk.shape
    bn = 1000
    n_blk = n_nodes // bn
    grid = (n_cb, n_blk)

    def body(h_ref, b_ref, sum_ref, max_ref, cnt_ref):
        cb = pl.program_id(0)
        i = pl.program_id(1)
        hb = h_ref[0]
        bi = b_ref[0]

        @pl.when(i == 0)
        def _():
            sum_ref[...] = jnp.zeros_like(sum_ref)
            max_ref[...] = jnp.full_like(max_ref, -jnp.inf)

        @pl.when(jnp.logical_and(i == 0, cb == 0))
        def _():
            cnt_ref[...] = jnp.zeros_like(cnt_ref)

        masks = [(bi == b).astype(F32) for b in range(n_batch)]
        maskf = jnp.concatenate(masks, axis=1)
        sums = lax.dot_general(maskf, hb, (((0,), (0,)), ((), ())),
                               preferred_element_type=F32)
        sum_ref[0] += sums
        hms = []
        for b in range(n_batch):
            mb = bi == b
            hms.append(jnp.max(jnp.where(mb, hb, -jnp.inf), axis=0)[None])
        max_ref[0] = jnp.maximum(max_ref[0], jnp.concatenate(hms, axis=0))

        @pl.when(cb == 0)
        def _():
            cnt_ref[...] += jnp.sum(maskf, axis=0)[:, None]

    return pl.pallas_call(
        body,
        grid=grid,
        in_specs=[
            pl.BlockSpec((1, bn, cbw), lambda cb, i: (cb, i, 0)),
            pl.BlockSpec((1, bn, 1), lambda cb, i: (i, 0, 0)),
        ],
        out_specs=[
            pl.BlockSpec((1, n_batch, cbw), lambda cb, i: (cb, 0, 0)),
            pl.BlockSpec((1, n_batch, cbw), lambda cb, i: (cb, 0, 0)),
            pl.BlockSpec((n_batch, cbw), lambda cb, i: (0, 0)),
        ],
        out_shape=[
            jax.ShapeDtypeStruct((n_cb, n_batch, cbw), F32),
            jax.ShapeDtypeStruct((n_cb, n_batch, cbw), F32),
            jax.ShapeDtypeStruct((n_batch, cbw), F32),
        ],
    )(h_blk, bidx3)


def _tc_head(sums, maxs, cnt, w1m, w1x, b1, w2, b2):
    n_cb, n_batch, cbw = sums.shape
    hid = w1m.shape[2]
    dout = w2.shape[1]

    def body(s_ref, m_ref, c_ref, w1m_ref, w1x_ref, b1_ref, w2_ref, b2_ref,
             o_ref):
        cntc = jnp.maximum(c_ref[:, 0:1], 1.0)
        acc = jnp.broadcast_to(b1_ref[...], (n_batch, hid))
        for k in range(n_cb):
            xm = s_ref[k] / cntc
            acc += jnp.dot(xm, w1m_ref[k], preferred_element_type=F32)
            mx = jnp.where(jnp.isfinite(m_ref[k]), m_ref[k], 0.0)
            acc += jnp.dot(mx, w1x_ref[k], preferred_element_type=F32)
        hdn = jnp.maximum(acc, 0.0)
        out = jnp.dot(hdn, w2_ref[...], preferred_element_type=F32)
        out += b2_ref[...]
        o_ref[...] = jnp.maximum(out, 0.0)

    return pl.pallas_call(
        body,
        out_shape=jax.ShapeDtypeStruct((n_batch, dout), F32),
    )(sums, maxs, cnt, w1m, w1x, b1, w2, b2)


def kernel(x, edge_index, batch_idx, conv_Ws, skip_Ws, mlp_W1, mlp_b1,
           mlp_W2, mlp_b2):
    n_nodes, cin0 = x.shape
    n_batch = 8
    e = edge_index.shape[1]
    src, dst = edge_index[0], edge_index[1]

    e_step = NUM_SC * NUM_TILES * K_EDGE
    e_pad = e + (-e) % e_step
    if e_pad != e:
        src = jnp.concatenate([src, jnp.zeros((e_pad - e,), jnp.int32)])
        dst = jnp.concatenate([dst, jnp.full((e_pad - e,), n_nodes, jnp.int32)])

    h = x.reshape(1, n_nodes, cin0)
    deg_pair = _deg_sc(dst, n_nodes)
    for w, ws in zip(conv_Ws, skip_Ws):
        cin, cout = w.shape
        cip, cop = max(128, cin), max(128, cout)
        if (cip, cop) != (cin, cout):
            w = jnp.zeros((cip, cop), F32).at[:cin, :cout].set(w)
            ws = jnp.zeros((cip, cop), F32).at[:cin, :cout].set(ws)
        gpair = _seg_sum_sc(h, src, dst)
        h = _tc_layer(gpair, h, deg_pair, w, ws)

    bidx3 = batch_idx.reshape(n_nodes // 1000, 1000, 1)
    sums, maxs, cnt = _tc_pool(h, bidx3, n_batch)

    hid = mlp_W1.shape[1]
    n_cb = h.shape[0]
    cbw = h.shape[2]
    w1m = mlp_W1[: n_cb * cbw].reshape(n_cb, cbw, hid)
    w1x = mlp_W1[n_cb * cbw:].reshape(n_cb, cbw, hid)
    return _tc_head(sums, maxs, cnt, w1m, w1x, mlp_b1.reshape(1, hid),
                    mlp_W2, mlp_b2.reshape(1, mlp_W2.shape[1]))


# speedup vs baseline: 2.3289x; 1.2287x over previous
import functools

import jax
import jax.numpy as jnp
from jax import lax
from jax.experimental import pallas as pl
from jax.experimental.pallas import tpu as pltpu
from jax.experimental.pallas import tpu_sc as plsc

F32 = jnp.float32
NUM_SC = 2
NUM_TILES = 16
K_EDGE = 128


def _seg_sum_sc(h_blk, src3, dst3):
    n_cb, n_nodes, cbw = h_blk.shape
    n_rows_e = src3.shape[0]
    e_tile_rows = n_rows_e // (NUM_SC * NUM_TILES)
    n_batches = e_tile_rows
    assert n_batches % 2 == 0
    nacc = n_nodes + 1
    nacc += (-nacc) % (NUM_TILES * 8)
    rows_per_tile = nacc // NUM_TILES
    chunks = []
    off = 0
    while off < rows_per_tile:
        ln = min(128, rows_per_tile - off)
        chunks.append((off, ln))
        off += ln

    mesh = plsc.VectorSubcoreMesh(core_axis_name="c", subcore_axis_name="s")
    out_type = jax.ShapeDtypeStruct((NUM_SC, n_cb, nacc, cbw), F32)
    scratch = [
        pltpu.VMEM((n_batches, 1, K_EDGE), jnp.int32),
        pltpu.VMEM((n_batches, 1, K_EDGE), jnp.int32),
        pltpu.VMEM((K_EDGE, cbw), F32),
        pltpu.VMEM((K_EDGE, cbw), F32),
        pltpu.VMEM_SHARED((nacc, cbw), F32),
        pltpu.SemaphoreType.DMA,
        pltpu.SemaphoreType.DMA,
    ]

    def body(h_hbm, src_hbm, dst_hbm, out_hbm, sidx, didx, buf0, buf1,
             acc, sem0, sem1):
        cid = lax.axis_index("c")
        sid = lax.axis_index("s")
        tile = cid * NUM_TILES + sid
        rbase = sid * rows_per_tile
        erow0 = tile * e_tile_rows

        pltpu.sync_copy(src_hbm.at[pl.ds(erow0, n_batches)], sidx)
        pltpu.sync_copy(dst_hbm.at[pl.ds(erow0, n_batches)], didx)

        z16 = jnp.zeros((16,), F32)
        for cb in range(n_cb):
            def zrow(r, _):
                for j in range(cbw // 16):
                    buf0[r, pl.ds(j * 16, 16)] = z16
                return 0
            lax.fori_loop(0, K_EDGE, zrow, 0)
            for c0, ln in chunks:
                pltpu.sync_copy(buf0.at[pl.ds(0, ln)],
                                acc.at[pl.ds(rbase + c0, ln)])
            plsc.subcore_barrier()

            tbl = h_hbm.at[cb]
            pltpu.async_copy(tbl.at[sidx.at[0, 0]], buf0, sem0)

            def pair(u, _):
                t0 = u * 2
                pltpu.async_copy(tbl.at[sidx.at[t0 + 1, 0]], buf1, sem1)
                pltpu.make_async_copy(tbl.at[pl.ds(0, K_EDGE)], buf0,
                                      sem0).wait()
                pltpu.sync_copy(buf0, acc.at[didx.at[t0, 0]], add=True)

                @pl.when(t0 + 2 < n_batches)
                def _():
                    pltpu.async_copy(tbl.at[sidx.at[t0 + 2, 0]], buf0, sem0)
                pltpu.make_async_copy(tbl.at[pl.ds(0, K_EDGE)], buf1,
                                      sem1).wait()
                pltpu.sync_copy(buf1, acc.at[didx.at[t0 + 1, 0]], add=True)
                return 0
            lax.fori_loop(0, n_batches // 2, pair, 0)
            plsc.subcore_barrier()

            for c0, ln in chunks:
                r0 = rbase + c0
                pltpu.sync_copy(acc.at[pl.ds(r0, ln)], buf0.at[pl.ds(0, ln)])
                pltpu.sync_copy(buf0.at[pl.ds(0, ln)],
                                out_hbm.at[cid, cb].at[pl.ds(r0, ln)])
            plsc.subcore_barrier()

    f = pl.kernel(body, out_type=out_type, mesh=mesh, scratch_types=scratch)
    return f(h_blk, src3, dst3)


def _deg_sc(dst3, n_nodes):
    n_rows_e = dst3.shape[0]
    e_tile_rows = n_rows_e // (NUM_SC * NUM_TILES)
    n_batches = e_tile_rows
    nacc = n_nodes + 1
    nacc += (-nacc) % (NUM_TILES * 8)
    rows_per_tile = nacc // NUM_TILES
    chunks = []
    off = 0
    while off < rows_per_tile:
        ln = min(128, rows_per_tile - off)
        chunks.append((off, ln))
        off += ln

    mesh = plsc.VectorSubcoreMesh(core_axis_name="c", subcore_axis_name="s")
    out_type = jax.ShapeDtypeStruct((NUM_SC, nacc, 128), F32)
    scratch = [
        pltpu.VMEM((n_batches, 1, K_EDGE), jnp.int32),
        pltpu.VMEM((K_EDGE, 128), F32),
        pltpu.VMEM((128, 128), F32),
        pltpu.VMEM_SHARED((nacc, 128), F32),
    ]

    def body(dst_hbm, deg_hbm, didx, ones16, dzbuf, dacc):
        cid = lax.axis_index("c")
        sid = lax.axis_index("s")
        tile = cid * NUM_TILES + sid
        rbase = sid * rows_per_tile
        erow0 = tile * e_tile_rows

        pltpu.sync_copy(dst_hbm.at[pl.ds(erow0, n_batches)], didx)

        z16 = jnp.zeros((16,), F32)
        hot = jnp.where(lax.iota(jnp.int32, 16) == 0, 1.0, 0.0).astype(F32)
        def orow(r, _):
            ones16[r, pl.ds(0, 16)] = hot
            for j in range(1, 8):
                ones16[r, pl.ds(j * 16, 16)] = z16
            for j in range(8):
                dzbuf[r, pl.ds(j * 16, 16)] = z16
            return 0
        lax.fori_loop(0, K_EDGE, orow, 0)

        for c0, ln in chunks:
            pltpu.sync_copy(dzbuf.at[pl.ds(0, ln)],
                            dacc.at[pl.ds(rbase + c0, ln)])
        plsc.subcore_barrier()

        def step(t, _):
            pltpu.sync_copy(ones16, dacc.at[didx.at[t, 0]], add=True)
            return 0
        lax.fori_loop(0, n_batches, step, 0)
        plsc.subcore_barrier()

        for c0, ln in chunks:
            r0 = rbase + c0
            pltpu.sync_copy(dacc.at[pl.ds(r0, ln)], dzbuf.at[pl.ds(0, ln)])
            pltpu.sync_copy(dzbuf.at[pl.ds(0, ln)],
                            deg_hbm.at[cid].at[pl.ds(r0, ln)])
        plsc.subcore_barrier()

    f = pl.kernel(body, out_type=out_type, mesh=mesh, scratch_types=scratch)
    return f(dst3)


def _tc_layer(gpair, h_blk, deg_pair, w, ws):
    n_in, n_nodes, cbin = h_blk.shape
    cin, cout = w.shape
    cbout = min(cout, 128)
    n_out = cout // cbout
    bn = 1000
    n_blk = n_nodes // bn
    grid = (n_out, n_blk, n_in)

    def body(g_ref, h_ref, d_ref, w_ref, ws_ref, o_ref):
        k = pl.program_id(2)
        deg = d_ref[0, :, 0:1] + d_ref[1, :, 0:1]
        invd = 1.0 / jnp.maximum(deg, 1.0)
        g = (g_ref[0, 0] + g_ref[1, 0]) * invd
        part = jnp.dot(g, w_ref[...], preferred_element_type=F32)
        part += jnp.dot(h_ref[0], ws_ref[...], preferred_element_type=F32)

        @pl.when(k == 0)
        def _():
            o_ref[0] = part

        @pl.when(k != 0)
        def _():
            o_ref[0] += part

        @pl.when(k == n_in - 1)
        def _():
            o_ref[0] = jnp.maximum(o_ref[0], 0.0)

    return pl.pallas_call(
        body,
        grid=grid,
        in_specs=[
            pl.BlockSpec((2, 1, bn, cbin), lambda o, i, k: (0, k, i, 0)),
            pl.BlockSpec((1, bn, cbin), lambda o, i, k: (k, i, 0)),
            pl.BlockSpec((2, bn, 128), lambda o, i, k: (0, i, 0)),
            pl.BlockSpec((cbin, cbout), lambda o, i, k: (k, o)),
            pl.BlockSpec((cbin, cbout), lambda o, i, k: (k, o)),
        ],
        out_specs=pl.BlockSpec((1, bn, cbout), lambda o, i, k: (o, i, 0)),
        out_shape=jax.ShapeDtypeStruct((n_out, n_nodes, cbout), F32),
    )(gpair, h_blk, deg_pair, w, ws)


def _tc_pool(h_blk, bidx3, n_batch):
    n_cb, n_nodes, cbw = h_blk.shape
    bn = 1000
    n_blk = n_nodes // bn
    grid = (n_cb, n_blk)

    def body(h_ref, b_ref, sum_ref, max_ref, cnt_ref):
        cb = pl.program_id(0)
        i = pl.program_id(1)
        hb = h_ref[0]
        bi = b_ref[0]

        @pl.when(i == 0)
        def _():
            sum_ref[...] = jnp.zeros_like(sum_ref)
            max_ref[...] = jnp.full_like(max_ref, -jnp.inf)

        @pl.when(jnp.logical_and(i == 0, cb == 0))
        def _():
            cnt_ref[...] = jnp.zeros_like(cnt_ref)

        masks = [(bi == b).astype(F32) for b in range(n_batch)]
        maskf = jnp.concatenate(masks, axis=1)
        sums = lax.dot_general(maskf, hb, (((0,), (0,)), ((), ())),
                               preferred_element_type=F32)
        sum_ref[0] += sums
        hms = []
        for b in range(n_batch):
            mb = bi == b
            hms.append(jnp.max(jnp.where(mb, hb, -jnp.inf), axis=0)[None])
        max_ref[0] = jnp.maximum(max_ref[0], jnp.concatenate(hms, axis=0))

        @pl.when(cb == 0)
        def _():
            cnt_ref[...] += jnp.sum(maskf, axis=0)[:, None]

    return pl.pallas_call(
        body,
        grid=grid,
        in_specs=[
            pl.BlockSpec((1, bn, cbw), lambda cb, i: (cb, i, 0)),
            pl.BlockSpec((1, bn, 1), lambda cb, i: (i, 0, 0)),
        ],
        out_specs=[
            pl.BlockSpec((1, n_batch, cbw), lambda cb, i: (cb, 0, 0)),
            pl.BlockSpec((1, n_batch, cbw), lambda cb, i: (cb, 0, 0)),
            pl.BlockSpec((n_batch, cbw), lambda cb, i: (0, 0)),
        ],
        out_shape=[
            jax.ShapeDtypeStruct((n_cb, n_batch, cbw), F32),
            jax.ShapeDtypeStruct((n_cb, n_batch, cbw), F32),
            jax.ShapeDtypeStruct((n_batch, cbw), F32),
        ],
    )(h_blk, bidx3)


def _tc_head(sums, maxs, cnt, w1m, w1x, b1, w2, b2):
    n_cb, n_batch, cbw = sums.shape
    hid = w1m.shape[2]
    dout = w2.shape[1]

    def body(s_ref, m_ref, c_ref, w1m_ref, w1x_ref, b1_ref, w2_ref, b2_ref,
             o_ref):
        cntc = jnp.maximum(c_ref[:, 0:1], 1.0)
        acc = jnp.broadcast_to(b1_ref[...], (n_batch, hid))
        for k in range(n_cb):
            xm = s_ref[k] / cntc
            acc += jnp.dot(xm, w1m_ref[k], preferred_element_type=F32)
            mx = jnp.where(jnp.isfinite(m_ref[k]), m_ref[k], 0.0)
            acc += jnp.dot(mx, w1x_ref[k], preferred_element_type=F32)
        hdn = jnp.maximum(acc, 0.0)
        out = jnp.dot(hdn, w2_ref[...], preferred_element_type=F32)
        out += b2_ref[...]
        o_ref[...] = jnp.maximum(out, 0.0)

    return pl.pallas_call(
        body,
        out_shape=jax.ShapeDtypeStruct((n_batch, dout), F32),
    )(sums, maxs, cnt, w1m, w1x, b1, w2, b2)


def kernel(x, edge_index, batch_idx, conv_Ws, skip_Ws, mlp_W1, mlp_b1,
           mlp_W2, mlp_b2):
    n_nodes, cin0 = x.shape
    n_batch = 8
    e = edge_index.shape[1]
    src, dst = edge_index[0], edge_index[1]

    e_step = NUM_SC * NUM_TILES * K_EDGE * 2
    e_pad = e + (-e) % e_step
    if e_pad != e:
        src = jnp.concatenate([src, jnp.zeros((e_pad - e,), jnp.int32)])
        dst = jnp.concatenate([dst, jnp.full((e_pad - e,), n_nodes, jnp.int32)])
    src = src.reshape(e_pad // K_EDGE, 1, K_EDGE)
    dst = dst.reshape(e_pad // K_EDGE, 1, K_EDGE)

    h = x.reshape(1, n_nodes, cin0)
    deg_pair = _deg_sc(dst, n_nodes)
    for w, ws in zip(conv_Ws, skip_Ws):
        cin, cout = w.shape
        cip, cop = max(128, cin), max(128, cout)
        if (cip, cop) != (cin, cout):
            w = jnp.zeros((cip, cop), F32).at[:cin, :cout].set(w)
            ws = jnp.zeros((cip, cop), F32).at[:cin, :cout].set(ws)
        gpair = _seg_sum_sc(h, src, dst)
        h = _tc_layer(gpair, h, deg_pair, w, ws)

    bidx3 = batch_idx.reshape(n_nodes // 1000, 1000, 1)
    sums, maxs, cnt = _tc_pool(h, bidx3, n_batch)

    hid = mlp_W1.shape[1]
    n_cb = h.shape[0]
    cbw = h.shape[2]
    w1m = mlp_W1[: n_cb * cbw].reshape(n_cb, cbw, hid)
    w1x = mlp_W1[n_cb * cbw:].reshape(n_cb, cbw, hid)
    return _tc_head(sums, maxs, cnt, w1m, w1x, mlp_b1.reshape(1, hid),
                    mlp_W2, mlp_b2.reshape(1, mlp_W2.shape[1]))
